# baseline (speedup 1.0000x reference)
import functools

import jax
import jax.numpy as jnp
import numpy as np
from jax.experimental import pallas as pl
from jax.experimental.pallas import tpu as pltpu

_BATCH = 64
_N_BINS = 128
_LENGTH = 4096
_MASK_VALUE = -80.0
_MAX_WIDTH = 32


def _rotl(x, r):
    return ((x << np.uint32(r)) | (x >> np.uint32(32 - r))).astype(np.uint32)


def _threefry2x32_pair(k1, k2, c1, c2):
    x = [c1.astype(np.uint32).copy(), c2.astype(np.uint32).copy()]
    rotations = [[13, 15, 26, 6], [17, 29, 16, 24]]
    ks = [np.uint32(k1), np.uint32(k2),
          np.uint32(np.uint32(k1) ^ np.uint32(k2) ^ np.uint32(0x1BD11BDA))]
    x[0] = (x[0] + ks[0]).astype(np.uint32)
    x[1] = (x[1] + ks[1]).astype(np.uint32)
    for i in range(5):
        for r in rotations[i % 2]:
            x[0] = (x[0] + x[1]).astype(np.uint32)
            x[1] = _rotl(x[1], r)
            x[1] = x[1] ^ x[0]
        x[0] = (x[0] + ks[(i + 1) % 3]).astype(np.uint32)
        x[1] = (x[1] + ks[(i + 2) % 3] + np.uint32(i + 1)).astype(np.uint32)
    return x[0], x[1]


def _np_uniform(k1, k2, n, minval, maxval):
    b1, b2 = _threefry2x32_pair(k1, k2, np.zeros(n, np.uint32),
                                np.arange(n, dtype=np.uint32))
    bits = b1 ^ b2
    fb = (bits >> np.uint32(9)) | np.uint32(0x3F800000)
    floats = fb.view(np.float32) - np.float32(1.0)
    r = (floats * np.float32(maxval - minval) + np.float32(minval)).astype(np.float32)
    return np.maximum(np.float32(minval), r)


def _mask_bounds() -> tuple[np.ndarray, np.ndarray]:
    b1, b2 = _threefry2x32_pair(np.uint32(0), np.uint32(42),
                                np.zeros(2, np.uint32),
                                np.arange(2, dtype=np.uint32))
    width = _np_uniform(b1[0], b2[0], _BATCH, 0.0, float(_MAX_WIDTH))
    ix = _np_uniform(b1[1], b2[1], _BATCH, 0.0, float(_N_BINS - _MAX_WIDTH))
    start = np.floor(ix).astype(np.int32)
    end = np.floor((ix + width).astype(np.float32)).astype(np.int32)
    return start, end


_START, _END = _mask_bounds()


def _row_lists():
    bins = np.arange(_N_BINS, dtype=np.int32)
    mask = ((bins[None, :] >= _START[:, None]) &
            (bins[None, :] < _END[:, None])).reshape(-1)
    rows = np.arange(_BATCH * _N_BINS, dtype=np.int32)

    def pad_split(r, k):
        per = -(-len(r) // (_NW * k)) * k
        total = per * _NW
        padded = np.concatenate([r, r[:total - len(r)]])
        return padded.reshape(_NW, per // k, k).astype(np.int32)

    return pad_split(rows[~mask], _K), pad_split(rows[mask], _KF)


_NW = 32
_NSUB = 16
_K = 6
_KF = 4
_NBUF = 3
_CIDX, _FIDX = _row_lists()
_GC = _CIDX.shape[1]
_GM = _FIDX.shape[1]


def _sc_body(x_ref, cidx_hbm, fidx_hbm, out_ref,
             cidx_v, fidx_v, buf0, buf1, buf2, fillbuf,
             gsem0, gsem1, gsem2, ssem0, ssem1, ssem2, fsem):
    from jax import lax
    wid = lax.axis_index("c") * _NSUB + lax.axis_index("s")
    pltpu.sync_copy(cidx_hbm.at[wid], cidx_v)
    pltpu.sync_copy(fidx_hbm.at[wid], fidx_v)

    n16 = _LENGTH // 16

    def fill_row(t, c):
        i = t // n16
        j = t % n16
        fillbuf[i, pl.ds(pl.multiple_of(j * 16, 16), 16)] = jnp.full(
            (16,), _MASK_VALUE, jnp.float32)
        return c

    lax.fori_loop(0, _KF * n16, fill_row, 0)

    fills = [pltpu.async_copy(fillbuf, out_ref.at[fidx_v.at[g]], fsem)
             for g in range(_GM)]

    bufs = (buf0, buf1, buf2)
    gsems = (gsem0, gsem1, gsem2)
    ssems = (ssem0, ssem1, ssem2)
    gathers = [None, None, None]
    scatters = [None, None, None]
    gathers[0] = pltpu.async_copy(x_ref.at[cidx_v.at[0]], bufs[0], gsems[0])
    for g in range(_GC):
        b = g % _NBUF
        if g + 1 < _GC:
            nb = (g + 1) % _NBUF
            if scatters[nb] is not None:
                scatters[nb].wait()
                scatters[nb] = None
            gathers[nb] = pltpu.async_copy(x_ref.at[cidx_v.at[g + 1]],
                                           bufs[nb], gsems[nb])
        gathers[b].wait()
        scatters[b] = pltpu.async_copy(bufs[b], out_ref.at[cidx_v.at[g]],
                                       ssems[b])
    for s in scatters:
        if s is not None:
            s.wait()
    for f in fills:
        f.wait()


def _kernel_sc(x):
    from jax.experimental.pallas import tpu_sc as plsc
    mesh = plsc.VectorSubcoreMesh(core_axis_name="c", subcore_axis_name="s")
    x2 = x.reshape(_BATCH * _N_BINS, _LENGTH)
    run = functools.partial(
        pl.kernel,
        out_type=jax.ShapeDtypeStruct((_BATCH * _N_BINS, _LENGTH), x.dtype),
        mesh=mesh,
        scratch_types=[
            pltpu.VMEM((_GC, _K), jnp.int32),
            pltpu.VMEM((_GM, _KF), jnp.int32),
            pltpu.VMEM((_K, _LENGTH), jnp.float32),
            pltpu.VMEM((_K, _LENGTH), jnp.float32),
            pltpu.VMEM((_K, _LENGTH), jnp.float32),
            pltpu.VMEM((_KF, _LENGTH), jnp.float32),
            pltpu.SemaphoreType.DMA,
            pltpu.SemaphoreType.DMA,
            pltpu.SemaphoreType.DMA,
            pltpu.SemaphoreType.DMA,
            pltpu.SemaphoreType.DMA,
            pltpu.SemaphoreType.DMA,
            pltpu.SemaphoreType.DMA,
        ],
    )(_sc_body)
    out = run(x2, jnp.asarray(_CIDX), jnp.asarray(_FIDX))
    return out.reshape(_BATCH, _N_BINS, _LENGTH)


_GROUP = 8
_NGRP = _N_BINS // _GROUP


def _src_groups() -> np.ndarray:
    src = np.empty(_BATCH * _NGRP, np.int32)
    for b in range(_BATCH):
        s, e = int(_START[b]), int(_END[b])
        prev = None
        for g in range(_NGRP):
            lo, hi = g * _GROUP, (g + 1) * _GROUP
            full = s <= lo and hi <= e
            if not full:
                mine = b * _NGRP + g
            elif prev is not None:
                mine = prev
            else:
                mine = b * _NGRP + _NGRP - 1
            src[b * _NGRP + g] = mine
            prev = mine
    return src


def _mask_kernel(srctab_ref, st_ref, en_ref, x_ref, o_ref):
    i = pl.program_id(0)
    b = i // _NGRP
    g = i % _NGRP
    row = g * _GROUP + jax.lax.broadcasted_iota(jnp.int32, (1, _GROUP, 1), 1)
    msk = (row >= st_ref[b]) & (row < en_ref[b])
    o_ref[...] = jnp.where(msk, jnp.float32(_MASK_VALUE), x_ref[...])


_SRCTAB = _src_groups()


def _kernel_tc(x):
    x2 = x.reshape(_BATCH * _NGRP, _GROUP, _LENGTH)
    out = pl.pallas_call(
        _mask_kernel,
        grid_spec=pltpu.PrefetchScalarGridSpec(
            num_scalar_prefetch=3,
            grid=(_BATCH * _NGRP,),
            in_specs=[pl.BlockSpec((1, _GROUP, _LENGTH),
                                   lambda i, tab, st, en: (tab[i], 0, 0))],
            out_specs=pl.BlockSpec((1, _GROUP, _LENGTH),
                                   lambda i, tab, st, en: (i, 0, 0)),
        ),
        out_shape=jax.ShapeDtypeStruct((_BATCH * _NGRP, _GROUP, _LENGTH),
                                       x.dtype),
    )(jnp.asarray(_SRCTAB), jnp.asarray(_START), jnp.asarray(_END), x2)
    return out.reshape(_BATCH, _N_BINS, _LENGTH)


_MAXW = 32


def _dma_body(x_ref, o_ref, fillbuf, csem, fsem):
    fillbuf[...] = jnp.full((_MAXW, _LENGTH // 128, 128), _MASK_VALUE,
                            jnp.float32)
    copies = []
    fills = []
    for b in range(_BATCH):
        s, e = int(_START[b]), int(_END[b])
        r0 = b * _N_BINS
        if s > 0:
            copies.append(pltpu.make_async_copy(
                x_ref.at[pl.ds(r0, s)], o_ref.at[pl.ds(r0, s)], csem))
        if e < _N_BINS:
            copies.append(pltpu.make_async_copy(
                x_ref.at[pl.ds(r0 + e, _N_BINS - e)],
                o_ref.at[pl.ds(r0 + e, _N_BINS - e)], csem))
        if e > s:
            fills.append(pltpu.make_async_copy(
                fillbuf.at[pl.ds(0, e - s)],
                o_ref.at[pl.ds(r0 + s, e - s)], fsem))
    for c in copies:
        c.start()
    for f in fills:
        f.start()
    for c in copies:
        c.wait()
    for f in fills:
        f.wait()


def _kernel_dma(x):
    x2 = x.reshape(_BATCH * _N_BINS, _LENGTH // 128, 128)
    out = pl.pallas_call(
        _dma_body,
        in_specs=[pl.BlockSpec(memory_space=pl.ANY)],
        out_specs=pl.BlockSpec(memory_space=pl.ANY),
        out_shape=jax.ShapeDtypeStruct(
            (_BATCH * _N_BINS, _LENGTH // 128, 128), x.dtype),
        scratch_shapes=[
            pltpu.VMEM((_MAXW, _LENGTH // 128, 128), jnp.float32),
            pltpu.SemaphoreType.DMA,
            pltpu.SemaphoreType.DMA,
        ],
    )(x2)
    return out.reshape(_BATCH, _N_BINS, _LENGTH)


@jax.jit
def kernel(x):
    return _kernel_dma(x)


# speedup vs baseline: 10.3220x; 10.3220x over previous
import functools

import jax
import jax.numpy as jnp
import numpy as np
from jax.experimental import pallas as pl
from jax.experimental.pallas import tpu as pltpu

_BATCH = 64
_N_BINS = 128
_LENGTH = 4096
_MASK_VALUE = -80.0
_MAX_WIDTH = 32


def _rotl(x, r):
    return ((x << np.uint32(r)) | (x >> np.uint32(32 - r))).astype(np.uint32)


def _threefry2x32_pair(k1, k2, c1, c2):
    x = [c1.astype(np.uint32).copy(), c2.astype(np.uint32).copy()]
    rotations = [[13, 15, 26, 6], [17, 29, 16, 24]]
    ks = [np.uint32(k1), np.uint32(k2),
          np.uint32(np.uint32(k1) ^ np.uint32(k2) ^ np.uint32(0x1BD11BDA))]
    x[0] = (x[0] + ks[0]).astype(np.uint32)
    x[1] = (x[1] + ks[1]).astype(np.uint32)
    for i in range(5):
        for r in rotations[i % 2]:
            x[0] = (x[0] + x[1]).astype(np.uint32)
            x[1] = _rotl(x[1], r)
            x[1] = x[1] ^ x[0]
        x[0] = (x[0] + ks[(i + 1) % 3]).astype(np.uint32)
        x[1] = (x[1] + ks[(i + 2) % 3] + np.uint32(i + 1)).astype(np.uint32)
    return x[0], x[1]


def _np_uniform(k1, k2, n, minval, maxval):
    b1, b2 = _threefry2x32_pair(k1, k2, np.zeros(n, np.uint32),
                                np.arange(n, dtype=np.uint32))
    bits = b1 ^ b2
    fb = (bits >> np.uint32(9)) | np.uint32(0x3F800000)
    floats = fb.view(np.float32) - np.float32(1.0)
    r = (floats * np.float32(maxval - minval) + np.float32(minval)).astype(np.float32)
    return np.maximum(np.float32(minval), r)


def _mask_bounds() -> tuple[np.ndarray, np.ndarray]:
    b1, b2 = _threefry2x32_pair(np.uint32(0), np.uint32(42),
                                np.zeros(2, np.uint32),
                                np.arange(2, dtype=np.uint32))
    width = _np_uniform(b1[0], b2[0], _BATCH, 0.0, float(_MAX_WIDTH))
    ix = _np_uniform(b1[1], b2[1], _BATCH, 0.0, float(_N_BINS - _MAX_WIDTH))
    start = np.floor(ix).astype(np.int32)
    end = np.floor((ix + width).astype(np.float32)).astype(np.int32)
    return start, end


_START, _END = _mask_bounds()


def _row_lists():
    bins = np.arange(_N_BINS, dtype=np.int32)
    mask = ((bins[None, :] >= _START[:, None]) &
            (bins[None, :] < _END[:, None])).reshape(-1)
    rows = np.arange(_BATCH * _N_BINS, dtype=np.int32)

    def pad_split(r, k):
        per = -(-len(r) // (_NW * k)) * k
        total = per * _NW
        padded = np.concatenate([r, r[:total - len(r)]])
        return padded.reshape(_NW, per // k, k).astype(np.int32)

    return pad_split(rows[~mask], _K), pad_split(rows[mask], _KF)


_NW = 32
_NSUB = 16
_K = 6
_KF = 4
_NBUF = 3
_CIDX, _FIDX = _row_lists()
_GC = _CIDX.shape[1]
_GM = _FIDX.shape[1]


def _sc_body(x_ref, cidx_hbm, fidx_hbm, out_ref,
             cidx_v, fidx_v, buf0, buf1, buf2, fillbuf,
             gsem0, gsem1, gsem2, ssem0, ssem1, ssem2, fsem):
    from jax import lax
    wid = lax.axis_index("c") * _NSUB + lax.axis_index("s")
    pltpu.sync_copy(cidx_hbm.at[wid], cidx_v)
    pltpu.sync_copy(fidx_hbm.at[wid], fidx_v)

    n16 = _LENGTH // 16

    def fill_row(t, c):
        i = t // n16
        j = t % n16
        fillbuf[i, pl.ds(pl.multiple_of(j * 16, 16), 16)] = jnp.full(
            (16,), _MASK_VALUE, jnp.float32)
        return c

    lax.fori_loop(0, _KF * n16, fill_row, 0)

    fills = [pltpu.async_copy(fillbuf, out_ref.at[fidx_v.at[g]], fsem)
             for g in range(_GM)]

    bufs = (buf0, buf1, buf2)
    gsems = (gsem0, gsem1, gsem2)
    ssems = (ssem0, ssem1, ssem2)
    gathers = [None, None, None]
    scatters = [None, None, None]
    gathers[0] = pltpu.async_copy(x_ref.at[cidx_v.at[0]], bufs[0], gsems[0])
    for g in range(_GC):
        b = g % _NBUF
        if g + 1 < _GC:
            nb = (g + 1) % _NBUF
            if scatters[nb] is not None:
                scatters[nb].wait()
                scatters[nb] = None
            gathers[nb] = pltpu.async_copy(x_ref.at[cidx_v.at[g + 1]],
                                           bufs[nb], gsems[nb])
        gathers[b].wait()
        scatters[b] = pltpu.async_copy(bufs[b], out_ref.at[cidx_v.at[g]],
                                       ssems[b])
    for s in scatters:
        if s is not None:
            s.wait()
    for f in fills:
        f.wait()


def _kernel_sc(x):
    from jax.experimental.pallas import tpu_sc as plsc
    mesh = plsc.VectorSubcoreMesh(core_axis_name="c", subcore_axis_name="s")
    x2 = x.reshape(_BATCH * _N_BINS, _LENGTH)
    run = functools.partial(
        pl.kernel,
        out_type=jax.ShapeDtypeStruct((_BATCH * _N_BINS, _LENGTH), x.dtype),
        mesh=mesh,
        scratch_types=[
            pltpu.VMEM((_GC, _K), jnp.int32),
            pltpu.VMEM((_GM, _KF), jnp.int32),
            pltpu.VMEM((_K, _LENGTH), jnp.float32),
            pltpu.VMEM((_K, _LENGTH), jnp.float32),
            pltpu.VMEM((_K, _LENGTH), jnp.float32),
            pltpu.VMEM((_KF, _LENGTH), jnp.float32),
            pltpu.SemaphoreType.DMA,
            pltpu.SemaphoreType.DMA,
            pltpu.SemaphoreType.DMA,
            pltpu.SemaphoreType.DMA,
            pltpu.SemaphoreType.DMA,
            pltpu.SemaphoreType.DMA,
            pltpu.SemaphoreType.DMA,
        ],
    )(_sc_body)
    out = run(x2, jnp.asarray(_CIDX), jnp.asarray(_FIDX))
    return out.reshape(_BATCH, _N_BINS, _LENGTH)


_GROUP = 8
_NGRP = _N_BINS // _GROUP


def _src_groups() -> np.ndarray:
    src = np.empty(_BATCH * _NGRP, np.int32)
    for b in range(_BATCH):
        s, e = int(_START[b]), int(_END[b])
        prev = None
        for g in range(_NGRP):
            lo, hi = g * _GROUP, (g + 1) * _GROUP
            full = s <= lo and hi <= e
            if not full:
                mine = b * _NGRP + g
            elif prev is not None:
                mine = prev
            else:
                mine = b * _NGRP + _NGRP - 1
            src[b * _NGRP + g] = mine
            prev = mine
    return src


def _mask_kernel(srctab_ref, st_ref, en_ref, x_ref, o_ref):
    i = pl.program_id(0)
    b = i // _NGRP
    g = i % _NGRP
    row = g * _GROUP + jax.lax.broadcasted_iota(jnp.int32, (1, _GROUP, 1), 1)
    msk = (row >= st_ref[b]) & (row < en_ref[b])
    o_ref[...] = jnp.where(msk, jnp.float32(_MASK_VALUE), x_ref[...])


_SRCTAB = _src_groups()


def _kernel_tc(x):
    x2 = x.reshape(_BATCH * _NGRP, _GROUP, _LENGTH)
    out = pl.pallas_call(
        _mask_kernel,
        grid_spec=pltpu.PrefetchScalarGridSpec(
            num_scalar_prefetch=3,
            grid=(_BATCH * _NGRP,),
            in_specs=[pl.BlockSpec((1, _GROUP, _LENGTH),
                                   lambda i, tab, st, en: (tab[i], 0, 0))],
            out_specs=pl.BlockSpec((1, _GROUP, _LENGTH),
                                   lambda i, tab, st, en: (i, 0, 0)),
        ),
        out_shape=jax.ShapeDtypeStruct((_BATCH * _NGRP, _GROUP, _LENGTH),
                                       x.dtype),
    )(jnp.asarray(_SRCTAB), jnp.asarray(_START), jnp.asarray(_END), x2)
    return out.reshape(_BATCH, _N_BINS, _LENGTH)


_MAXW = 32


def _dma_body(x_ref, o_ref, fillbuf, csem, fsem):
    fillbuf[...] = jnp.full((_MAXW, _LENGTH // 128, 128), _MASK_VALUE,
                            jnp.float32)
    copies = []
    fills = []
    for b in range(_BATCH):
        s, e = int(_START[b]), int(_END[b])
        r0 = b * _N_BINS
        if s > 0:
            copies.append(pltpu.make_async_copy(
                x_ref.at[pl.ds(r0, s)], o_ref.at[pl.ds(r0, s)], csem))
        if e < _N_BINS:
            copies.append(pltpu.make_async_copy(
                x_ref.at[pl.ds(r0 + e, _N_BINS - e)],
                o_ref.at[pl.ds(r0 + e, _N_BINS - e)], csem))
        if e > s:
            fills.append(pltpu.make_async_copy(
                fillbuf.at[pl.ds(0, e - s)],
                o_ref.at[pl.ds(r0 + s, e - s)], fsem))
    for c in copies:
        c.start()
    for f in fills:
        f.start()
    for c in copies:
        c.wait()
    for f in fills:
        f.wait()


def _kernel_dma(x):
    x2 = x.reshape(_BATCH * _N_BINS, _LENGTH // 128, 128)
    out = pl.pallas_call(
        _dma_body,
        in_specs=[pl.BlockSpec(memory_space=pl.ANY)],
        out_specs=pl.BlockSpec(memory_space=pl.ANY),
        out_shape=jax.ShapeDtypeStruct(
            (_BATCH * _N_BINS, _LENGTH // 128, 128), x.dtype),
        scratch_shapes=[
            pltpu.VMEM((_MAXW, _LENGTH // 128, 128), jnp.float32),
            pltpu.SemaphoreType.DMA,
            pltpu.SemaphoreType.DMA,
        ],
    )(x2)
    return out.reshape(_BATCH, _N_BINS, _LENGTH)


_CH = 32
_NBUF_TC = 6


def _chunks() -> list:
    out = []
    for b in range(_BATCH):
        s, e = int(_START[b]), int(_END[b])
        r0 = b * _N_BINS
        for lo, hi in ((r0, r0 + s), (r0 + e, r0 + _N_BINS)):
            r = lo
            while r < hi:
                n = min(_CH, hi - r)
                out.append((r, n))
                r += n
    return out


_CHUNKS = _chunks()


def _stage_body(x_ref, o_ref, b0, b1, b2, b3, b4, b5, fillbuf,
                i0, i1, i2, i3, i4, i5, o0, o1, o2, o3, o4, o5, fsem):
    bufs = (b0, b1, b2, b3, b4, b5)
    isems = (i0, i1, i2, i3, i4, i5)
    osems = (o0, o1, o2, o3, o4, o5)
    nc = len(_CHUNKS)

    fillbuf[...] = jnp.full((_MAXW, _LENGTH // 128, 128), _MASK_VALUE,
                            jnp.float32)
    fills = []
    for b in range(_BATCH):
        s, e = int(_START[b]), int(_END[b])
        if e > s:
            f = pltpu.make_async_copy(
                fillbuf.at[pl.ds(0, e - s)],
                o_ref.at[pl.ds(b * _N_BINS + s, e - s)], fsem)
            f.start()
            fills.append(f)

    def start_in(i):
        r, n = _CHUNKS[i]
        h = pltpu.make_async_copy(x_ref.at[pl.ds(r, n)],
                                  bufs[i % _NBUF_TC].at[pl.ds(0, n)],
                                  isems[i % _NBUF_TC])
        h.start()
        return h

    in_h = {}
    out_h = {}
    for i in range(min(_NBUF_TC - 2, nc)):
        in_h[i] = start_in(i)
    for i in range(nc):
        b = i % _NBUF_TC
        in_h.pop(i).wait()
        r, n = _CHUNKS[i]
        h = pltpu.make_async_copy(bufs[b].at[pl.ds(0, n)],
                                  o_ref.at[pl.ds(r, n)], osems[b])
        h.start()
        out_h[i] = h
        j = i + _NBUF_TC - 2
        if j < nc:
            prev = j - _NBUF_TC
            if prev >= 0:
                out_h.pop(prev).wait()
            in_h[j] = start_in(j)
    for h in out_h.values():
        h.wait()
    for f in fills:
        f.wait()


def _kernel_stage(x):
    x2 = x.reshape(_BATCH * _N_BINS, _LENGTH // 128, 128)
    buf = pltpu.VMEM((_CH, _LENGTH // 128, 128), jnp.float32)
    out = pl.pallas_call(
        _stage_body,
        in_specs=[pl.BlockSpec(memory_space=pl.ANY)],
        out_specs=pl.BlockSpec(memory_space=pl.ANY),
        out_shape=jax.ShapeDtypeStruct(
            (_BATCH * _N_BINS, _LENGTH // 128, 128), x.dtype),
        scratch_shapes=(
            [buf] * _NBUF_TC
            + [pltpu.VMEM((_MAXW, _LENGTH // 128, 128), jnp.float32)]
            + [pltpu.SemaphoreType.DMA] * (2 * _NBUF_TC + 1)
        ),
    )(x2)
    return out.reshape(_BATCH, _N_BINS, _LENGTH)


@jax.jit
def kernel(x):
    return _kernel_stage(x)


# speedup vs baseline: 19.7493x; 1.9133x over previous
import functools

import jax
import jax.numpy as jnp
import numpy as np
from jax.experimental import pallas as pl
from jax.experimental.pallas import tpu as pltpu

_BATCH = 64
_N_BINS = 128
_LENGTH = 4096
_MASK_VALUE = -80.0
_MAX_WIDTH = 32


def _rotl(x, r):
    return ((x << np.uint32(r)) | (x >> np.uint32(32 - r))).astype(np.uint32)


def _threefry2x32_pair(k1, k2, c1, c2):
    x = [c1.astype(np.uint32).copy(), c2.astype(np.uint32).copy()]
    rotations = [[13, 15, 26, 6], [17, 29, 16, 24]]
    ks = [np.uint32(k1), np.uint32(k2),
          np.uint32(np.uint32(k1) ^ np.uint32(k2) ^ np.uint32(0x1BD11BDA))]
    x[0] = (x[0] + ks[0]).astype(np.uint32)
    x[1] = (x[1] + ks[1]).astype(np.uint32)
    for i in range(5):
        for r in rotations[i % 2]:
            x[0] = (x[0] + x[1]).astype(np.uint32)
            x[1] = _rotl(x[1], r)
            x[1] = x[1] ^ x[0]
        x[0] = (x[0] + ks[(i + 1) % 3]).astype(np.uint32)
        x[1] = (x[1] + ks[(i + 2) % 3] + np.uint32(i + 1)).astype(np.uint32)
    return x[0], x[1]


def _np_uniform(k1, k2, n, minval, maxval):
    b1, b2 = _threefry2x32_pair(k1, k2, np.zeros(n, np.uint32),
                                np.arange(n, dtype=np.uint32))
    bits = b1 ^ b2
    fb = (bits >> np.uint32(9)) | np.uint32(0x3F800000)
    floats = fb.view(np.float32) - np.float32(1.0)
    r = (floats * np.float32(maxval - minval) + np.float32(minval)).astype(np.float32)
    return np.maximum(np.float32(minval), r)


def _mask_bounds() -> tuple[np.ndarray, np.ndarray]:
    b1, b2 = _threefry2x32_pair(np.uint32(0), np.uint32(42),
                                np.zeros(2, np.uint32),
                                np.arange(2, dtype=np.uint32))
    width = _np_uniform(b1[0], b2[0], _BATCH, 0.0, float(_MAX_WIDTH))
    ix = _np_uniform(b1[1], b2[1], _BATCH, 0.0, float(_N_BINS - _MAX_WIDTH))
    start = np.floor(ix).astype(np.int32)
    end = np.floor((ix + width).astype(np.float32)).astype(np.int32)
    return start, end


_START, _END = _mask_bounds()


def _row_lists():
    bins = np.arange(_N_BINS, dtype=np.int32)
    mask = ((bins[None, :] >= _START[:, None]) &
            (bins[None, :] < _END[:, None])).reshape(-1)
    rows = np.arange(_BATCH * _N_BINS, dtype=np.int32)

    def pad_split(r, k):
        per = -(-len(r) // (_NW * k)) * k
        total = per * _NW
        padded = np.concatenate([r, r[:total - len(r)]])
        return padded.reshape(_NW, per // k, k).astype(np.int32)

    return pad_split(rows[~mask], _K), pad_split(rows[mask], _KF)


_NW = 32
_NSUB = 16
_K = 6
_KF = 4
_NBUF = 3
_CIDX, _FIDX = _row_lists()
_GC = _CIDX.shape[1]
_GM = _FIDX.shape[1]


def _sc_body(x_ref, cidx_hbm, fidx_hbm, out_ref,
             cidx_v, fidx_v, buf0, buf1, buf2, fillbuf,
             gsem0, gsem1, gsem2, ssem0, ssem1, ssem2, fsem):
    from jax import lax
    wid = lax.axis_index("c") * _NSUB + lax.axis_index("s")
    pltpu.sync_copy(cidx_hbm.at[wid], cidx_v)
    pltpu.sync_copy(fidx_hbm.at[wid], fidx_v)

    n16 = _LENGTH // 16

    def fill_row(t, c):
        i = t // n16
        j = t % n16
        fillbuf[i, pl.ds(pl.multiple_of(j * 16, 16), 16)] = jnp.full(
            (16,), _MASK_VALUE, jnp.float32)
        return c

    lax.fori_loop(0, _KF * n16, fill_row, 0)

    fills = [pltpu.async_copy(fillbuf, out_ref.at[fidx_v.at[g]], fsem)
             for g in range(_GM)]

    bufs = (buf0, buf1, buf2)
    gsems = (gsem0, gsem1, gsem2)
    ssems = (ssem0, ssem1, ssem2)
    gathers = [None, None, None]
    scatters = [None, None, None]
    gathers[0] = pltpu.async_copy(x_ref.at[cidx_v.at[0]], bufs[0], gsems[0])
    for g in range(_GC):
        b = g % _NBUF
        if g + 1 < _GC:
            nb = (g + 1) % _NBUF
            if scatters[nb] is not None:
                scatters[nb].wait()
                scatters[nb] = None
            gathers[nb] = pltpu.async_copy(x_ref.at[cidx_v.at[g + 1]],
                                           bufs[nb], gsems[nb])
        gathers[b].wait()
        scatters[b] = pltpu.async_copy(bufs[b], out_ref.at[cidx_v.at[g]],
                                       ssems[b])
    for s in scatters:
        if s is not None:
            s.wait()
    for f in fills:
        f.wait()


def _kernel_sc(x):
    from jax.experimental.pallas import tpu_sc as plsc
    mesh = plsc.VectorSubcoreMesh(core_axis_name="c", subcore_axis_name="s")
    x2 = x.reshape(_BATCH * _N_BINS, _LENGTH)
    run = functools.partial(
        pl.kernel,
        out_type=jax.ShapeDtypeStruct((_BATCH * _N_BINS, _LENGTH), x.dtype),
        mesh=mesh,
        scratch_types=[
            pltpu.VMEM((_GC, _K), jnp.int32),
            pltpu.VMEM((_GM, _KF), jnp.int32),
            pltpu.VMEM((_K, _LENGTH), jnp.float32),
            pltpu.VMEM((_K, _LENGTH), jnp.float32),
            pltpu.VMEM((_K, _LENGTH), jnp.float32),
            pltpu.VMEM((_KF, _LENGTH), jnp.float32),
            pltpu.SemaphoreType.DMA,
            pltpu.SemaphoreType.DMA,
            pltpu.SemaphoreType.DMA,
            pltpu.SemaphoreType.DMA,
            pltpu.SemaphoreType.DMA,
            pltpu.SemaphoreType.DMA,
            pltpu.SemaphoreType.DMA,
        ],
    )(_sc_body)
    out = run(x2, jnp.asarray(_CIDX), jnp.asarray(_FIDX))
    return out.reshape(_BATCH, _N_BINS, _LENGTH)


_GROUP = 8
_NGRP = _N_BINS // _GROUP


def _src_groups() -> np.ndarray:
    src = np.empty(_BATCH * _NGRP, np.int32)
    for b in range(_BATCH):
        s, e = int(_START[b]), int(_END[b])
        prev = None
        for g in range(_NGRP):
            lo, hi = g * _GROUP, (g + 1) * _GROUP
            full = s <= lo and hi <= e
            if not full:
                mine = b * _NGRP + g
            elif prev is not None:
                mine = prev
            else:
                mine = b * _NGRP + _NGRP - 1
            src[b * _NGRP + g] = mine
            prev = mine
    return src


def _mask_kernel(srctab_ref, st_ref, en_ref, x_ref, o_ref):
    i = pl.program_id(0)
    b = i // _NGRP
    g = i % _NGRP
    row = g * _GROUP + jax.lax.broadcasted_iota(jnp.int32, (1, _GROUP, 1), 1)
    msk = (row >= st_ref[b]) & (row < en_ref[b])
    o_ref[...] = jnp.where(msk, jnp.float32(_MASK_VALUE), x_ref[...])


_SRCTAB = _src_groups()


def _kernel_tc(x):
    x2 = x.reshape(_BATCH * _NGRP, _GROUP, _LENGTH)
    out = pl.pallas_call(
        _mask_kernel,
        grid_spec=pltpu.PrefetchScalarGridSpec(
            num_scalar_prefetch=3,
            grid=(_BATCH * _NGRP,),
            in_specs=[pl.BlockSpec((1, _GROUP, _LENGTH),
                                   lambda i, tab, st, en: (tab[i], 0, 0))],
            out_specs=pl.BlockSpec((1, _GROUP, _LENGTH),
                                   lambda i, tab, st, en: (i, 0, 0)),
        ),
        out_shape=jax.ShapeDtypeStruct((_BATCH * _NGRP, _GROUP, _LENGTH),
                                       x.dtype),
    )(jnp.asarray(_SRCTAB), jnp.asarray(_START), jnp.asarray(_END), x2)
    return out.reshape(_BATCH, _N_BINS, _LENGTH)


_MAXW = 32


def _dma_body(x_ref, o_ref, fillbuf, csem, fsem):
    fillbuf[...] = jnp.full((_MAXW, _LENGTH // 128, 128), _MASK_VALUE,
                            jnp.float32)
    copies = []
    fills = []
    for b in range(_BATCH):
        s, e = int(_START[b]), int(_END[b])
        r0 = b * _N_BINS
        if s > 0:
            copies.append(pltpu.make_async_copy(
                x_ref.at[pl.ds(r0, s)], o_ref.at[pl.ds(r0, s)], csem))
        if e < _N_BINS:
            copies.append(pltpu.make_async_copy(
                x_ref.at[pl.ds(r0 + e, _N_BINS - e)],
                o_ref.at[pl.ds(r0 + e, _N_BINS - e)], csem))
        if e > s:
            fills.append(pltpu.make_async_copy(
                fillbuf.at[pl.ds(0, e - s)],
                o_ref.at[pl.ds(r0 + s, e - s)], fsem))
    for c in copies:
        c.start()
    for f in fills:
        f.start()
    for c in copies:
        c.wait()
    for f in fills:
        f.wait()


def _kernel_dma(x):
    x2 = x.reshape(_BATCH * _N_BINS, _LENGTH // 128, 128)
    out = pl.pallas_call(
        _dma_body,
        in_specs=[pl.BlockSpec(memory_space=pl.ANY)],
        out_specs=pl.BlockSpec(memory_space=pl.ANY),
        out_shape=jax.ShapeDtypeStruct(
            (_BATCH * _N_BINS, _LENGTH // 128, 128), x.dtype),
        scratch_shapes=[
            pltpu.VMEM((_MAXW, _LENGTH // 128, 128), jnp.float32),
            pltpu.SemaphoreType.DMA,
            pltpu.SemaphoreType.DMA,
        ],
    )(x2)
    return out.reshape(_BATCH, _N_BINS, _LENGTH)


_CH = 32
_NBUF_TC = 6


def _chunks() -> list:
    out = []
    for b in range(_BATCH):
        s, e = int(_START[b]), int(_END[b])
        r0 = b * _N_BINS
        for lo, hi in ((r0, r0 + s), (r0 + e, r0 + _N_BINS)):
            r = lo
            while r < hi:
                n = min(_CH, hi - r)
                out.append((r, n))
                r += n
    return out


_CHUNKS = _chunks()


def _stage_body(x_ref, o_ref, b0, b1, b2, b3, b4, b5, fillbuf,
                i0, i1, i2, i3, i4, i5, o0, o1, o2, o3, o4, o5, fsem):
    bufs = (b0, b1, b2, b3, b4, b5)
    isems = (i0, i1, i2, i3, i4, i5)
    osems = (o0, o1, o2, o3, o4, o5)
    nc = len(_CHUNKS)

    fillbuf[...] = jnp.full((_MAXW, _LENGTH // 128, 128), _MASK_VALUE,
                            jnp.float32)
    fills = []
    for b in range(_BATCH):
        s, e = int(_START[b]), int(_END[b])
        if e > s:
            f = pltpu.make_async_copy(
                fillbuf.at[pl.ds(0, e - s)],
                o_ref.at[pl.ds(b * _N_BINS + s, e - s)], fsem)
            f.start()
            fills.append(f)

    def start_in(i):
        r, n = _CHUNKS[i]
        h = pltpu.make_async_copy(x_ref.at[pl.ds(r, n)],
                                  bufs[i % _NBUF_TC].at[pl.ds(0, n)],
                                  isems[i % _NBUF_TC])
        h.start()
        return h

    in_h = {}
    out_h = {}
    for i in range(min(_NBUF_TC - 2, nc)):
        in_h[i] = start_in(i)
    for i in range(nc):
        b = i % _NBUF_TC
        in_h.pop(i).wait()
        r, n = _CHUNKS[i]
        h = pltpu.make_async_copy(bufs[b].at[pl.ds(0, n)],
                                  o_ref.at[pl.ds(r, n)], osems[b])
        h.start()
        out_h[i] = h
        j = i + _NBUF_TC - 2
        if j < nc:
            prev = j - _NBUF_TC
            if prev >= 0:
                out_h.pop(prev).wait()
            in_h[j] = start_in(j)
    for h in out_h.values():
        h.wait()
    for f in fills:
        f.wait()


def _kernel_stage(x):
    x2 = x.reshape(_BATCH * _N_BINS, _LENGTH // 128, 128)
    buf = pltpu.VMEM((_CH, _LENGTH // 128, 128), jnp.float32)
    out = pl.pallas_call(
        _stage_body,
        in_specs=[pl.BlockSpec(memory_space=pl.ANY)],
        out_specs=pl.BlockSpec(memory_space=pl.ANY),
        out_shape=jax.ShapeDtypeStruct(
            (_BATCH * _N_BINS, _LENGTH // 128, 128), x.dtype),
        scratch_shapes=(
            [buf] * _NBUF_TC
            + [pltpu.VMEM((_MAXW, _LENGTH // 128, 128), jnp.float32)]
            + [pltpu.SemaphoreType.DMA] * (2 * _NBUF_TC + 1)
        ),
    )(x2)
    return out.reshape(_BATCH, _N_BINS, _LENGTH)


_BLK = 1024


def _where_kernel(st_ref, en_ref, x_ref, o_ref):
    b = pl.program_id(0)
    row = jax.lax.broadcasted_iota(jnp.int32, (1, _N_BINS, 1), 1)
    msk = (row >= st_ref[b]) & (row < en_ref[b])
    o_ref[...] = jnp.where(msk, jnp.float32(_MASK_VALUE), x_ref[...])


def _kernel_where(x, blk=_BLK):
    return pl.pallas_call(
        _where_kernel,
        grid_spec=pltpu.PrefetchScalarGridSpec(
            num_scalar_prefetch=2,
            grid=(_BATCH, _LENGTH // blk),
            in_specs=[pl.BlockSpec((1, _N_BINS, blk),
                                   lambda b, l, st, en: (b, 0, l))],
            out_specs=pl.BlockSpec((1, _N_BINS, blk),
                                   lambda b, l, st, en: (b, 0, l)),
        ),
        out_shape=jax.ShapeDtypeStruct((_BATCH, _N_BINS, _LENGTH), x.dtype),
    )(jnp.asarray(_START), jnp.asarray(_END), x)


@jax.jit
def kernel(x):
    return _kernel_where(x)


# speedup vs baseline: 43.8173x; 2.2187x over previous
import functools

import jax
import jax.numpy as jnp
import numpy as np
from jax.experimental import pallas as pl
from jax.experimental.pallas import tpu as pltpu

_BATCH = 64
_N_BINS = 128
_LENGTH = 4096
_MASK_VALUE = -80.0
_MAX_WIDTH = 32


def _rotl(x, r):
    return ((x << np.uint32(r)) | (x >> np.uint32(32 - r))).astype(np.uint32)


def _threefry2x32_pair(k1, k2, c1, c2):
    x = [c1.astype(np.uint32).copy(), c2.astype(np.uint32).copy()]
    rotations = [[13, 15, 26, 6], [17, 29, 16, 24]]
    ks = [np.uint32(k1), np.uint32(k2),
          np.uint32(np.uint32(k1) ^ np.uint32(k2) ^ np.uint32(0x1BD11BDA))]
    x[0] = (x[0] + ks[0]).astype(np.uint32)
    x[1] = (x[1] + ks[1]).astype(np.uint32)
    for i in range(5):
        for r in rotations[i % 2]:
            x[0] = (x[0] + x[1]).astype(np.uint32)
            x[1] = _rotl(x[1], r)
            x[1] = x[1] ^ x[0]
        x[0] = (x[0] + ks[(i + 1) % 3]).astype(np.uint32)
        x[1] = (x[1] + ks[(i + 2) % 3] + np.uint32(i + 1)).astype(np.uint32)
    return x[0], x[1]


def _np_uniform(k1, k2, n, minval, maxval):
    b1, b2 = _threefry2x32_pair(k1, k2, np.zeros(n, np.uint32),
                                np.arange(n, dtype=np.uint32))
    bits = b1 ^ b2
    fb = (bits >> np.uint32(9)) | np.uint32(0x3F800000)
    floats = fb.view(np.float32) - np.float32(1.0)
    r = (floats * np.float32(maxval - minval) + np.float32(minval)).astype(np.float32)
    return np.maximum(np.float32(minval), r)


def _mask_bounds() -> tuple[np.ndarray, np.ndarray]:
    b1, b2 = _threefry2x32_pair(np.uint32(0), np.uint32(42),
                                np.zeros(2, np.uint32),
                                np.arange(2, dtype=np.uint32))
    width = _np_uniform(b1[0], b2[0], _BATCH, 0.0, float(_MAX_WIDTH))
    ix = _np_uniform(b1[1], b2[1], _BATCH, 0.0, float(_N_BINS - _MAX_WIDTH))
    start = np.floor(ix).astype(np.int32)
    end = np.floor((ix + width).astype(np.float32)).astype(np.int32)
    return start, end


_START, _END = _mask_bounds()


def _row_lists():
    bins = np.arange(_N_BINS, dtype=np.int32)
    mask = ((bins[None, :] >= _START[:, None]) &
            (bins[None, :] < _END[:, None])).reshape(-1)
    rows = np.arange(_BATCH * _N_BINS, dtype=np.int32)

    def pad_split(r, k):
        per = -(-len(r) // (_NW * k)) * k
        total = per * _NW
        padded = np.concatenate([r, r[:total - len(r)]])
        return padded.reshape(_NW, per // k, k).astype(np.int32)

    return pad_split(rows[~mask], _K), pad_split(rows[mask], _KF)


_NW = 32
_NSUB = 16
_K = 6
_KF = 4
_NBUF = 3
_CIDX, _FIDX = _row_lists()
_GC = _CIDX.shape[1]
_GM = _FIDX.shape[1]


def _sc_body(x_ref, cidx_hbm, fidx_hbm, out_ref,
             cidx_v, fidx_v, buf0, buf1, buf2, fillbuf,
             gsem0, gsem1, gsem2, ssem0, ssem1, ssem2, fsem):
    from jax import lax
    wid = lax.axis_index("c") * _NSUB + lax.axis_index("s")
    pltpu.sync_copy(cidx_hbm.at[wid], cidx_v)
    pltpu.sync_copy(fidx_hbm.at[wid], fidx_v)

    n16 = _LENGTH // 16

    def fill_row(t, c):
        i = t // n16
        j = t % n16
        fillbuf[i, pl.ds(pl.multiple_of(j * 16, 16), 16)] = jnp.full(
            (16,), _MASK_VALUE, jnp.float32)
        return c

    lax.fori_loop(0, _KF * n16, fill_row, 0)

    fills = [pltpu.async_copy(fillbuf, out_ref.at[fidx_v.at[g]], fsem)
             for g in range(_GM)]

    bufs = (buf0, buf1, buf2)
    gsems = (gsem0, gsem1, gsem2)
    ssems = (ssem0, ssem1, ssem2)
    gathers = [None, None, None]
    scatters = [None, None, None]
    gathers[0] = pltpu.async_copy(x_ref.at[cidx_v.at[0]], bufs[0], gsems[0])
    for g in range(_GC):
        b = g % _NBUF
        if g + 1 < _GC:
            nb = (g + 1) % _NBUF
            if scatters[nb] is not None:
                scatters[nb].wait()
                scatters[nb] = None
            gathers[nb] = pltpu.async_copy(x_ref.at[cidx_v.at[g + 1]],
                                           bufs[nb], gsems[nb])
        gathers[b].wait()
        scatters[b] = pltpu.async_copy(bufs[b], out_ref.at[cidx_v.at[g]],
                                       ssems[b])
    for s in scatters:
        if s is not None:
            s.wait()
    for f in fills:
        f.wait()


def _kernel_sc(x):
    from jax.experimental.pallas import tpu_sc as plsc
    mesh = plsc.VectorSubcoreMesh(core_axis_name="c", subcore_axis_name="s")
    x2 = x.reshape(_BATCH * _N_BINS, _LENGTH)
    run = functools.partial(
        pl.kernel,
        out_type=jax.ShapeDtypeStruct((_BATCH * _N_BINS, _LENGTH), x.dtype),
        mesh=mesh,
        scratch_types=[
            pltpu.VMEM((_GC, _K), jnp.int32),
            pltpu.VMEM((_GM, _KF), jnp.int32),
            pltpu.VMEM((_K, _LENGTH), jnp.float32),
            pltpu.VMEM((_K, _LENGTH), jnp.float32),
            pltpu.VMEM((_K, _LENGTH), jnp.float32),
            pltpu.VMEM((_KF, _LENGTH), jnp.float32),
            pltpu.SemaphoreType.DMA,
            pltpu.SemaphoreType.DMA,
            pltpu.SemaphoreType.DMA,
            pltpu.SemaphoreType.DMA,
            pltpu.SemaphoreType.DMA,
            pltpu.SemaphoreType.DMA,
            pltpu.SemaphoreType.DMA,
        ],
    )(_sc_body)
    out = run(x2, jnp.asarray(_CIDX), jnp.asarray(_FIDX))
    return out.reshape(_BATCH, _N_BINS, _LENGTH)


_GROUP = 8
_NGRP = _N_BINS // _GROUP


def _src_groups() -> np.ndarray:
    src = np.empty(_BATCH * _NGRP, np.int32)
    for b in range(_BATCH):
        s, e = int(_START[b]), int(_END[b])
        prev = None
        for g in range(_NGRP):
            lo, hi = g * _GROUP, (g + 1) * _GROUP
            full = s <= lo and hi <= e
            if not full:
                mine = b * _NGRP + g
            elif prev is not None:
                mine = prev
            else:
                mine = b * _NGRP + _NGRP - 1
            src[b * _NGRP + g] = mine
            prev = mine
    return src


def _mask_kernel(srctab_ref, st_ref, en_ref, x_ref, o_ref):
    i = pl.program_id(0)
    b = i // _NGRP
    g = i % _NGRP
    row = g * _GROUP + jax.lax.broadcasted_iota(jnp.int32, (1, _GROUP, 1), 1)
    msk = (row >= st_ref[b]) & (row < en_ref[b])
    o_ref[...] = jnp.where(msk, jnp.float32(_MASK_VALUE), x_ref[...])


_SRCTAB = _src_groups()


def _kernel_tc(x):
    x2 = x.reshape(_BATCH * _NGRP, _GROUP, _LENGTH)
    out = pl.pallas_call(
        _mask_kernel,
        grid_spec=pltpu.PrefetchScalarGridSpec(
            num_scalar_prefetch=3,
            grid=(_BATCH * _NGRP,),
            in_specs=[pl.BlockSpec((1, _GROUP, _LENGTH),
                                   lambda i, tab, st, en: (tab[i], 0, 0))],
            out_specs=pl.BlockSpec((1, _GROUP, _LENGTH),
                                   lambda i, tab, st, en: (i, 0, 0)),
        ),
        out_shape=jax.ShapeDtypeStruct((_BATCH * _NGRP, _GROUP, _LENGTH),
                                       x.dtype),
    )(jnp.asarray(_SRCTAB), jnp.asarray(_START), jnp.asarray(_END), x2)
    return out.reshape(_BATCH, _N_BINS, _LENGTH)


_MAXW = 32


def _dma_body(x_ref, o_ref, fillbuf, csem, fsem):
    fillbuf[...] = jnp.full((_MAXW, _LENGTH // 128, 128), _MASK_VALUE,
                            jnp.float32)
    copies = []
    fills = []
    for b in range(_BATCH):
        s, e = int(_START[b]), int(_END[b])
        r0 = b * _N_BINS
        if s > 0:
            copies.append(pltpu.make_async_copy(
                x_ref.at[pl.ds(r0, s)], o_ref.at[pl.ds(r0, s)], csem))
        if e < _N_BINS:
            copies.append(pltpu.make_async_copy(
                x_ref.at[pl.ds(r0 + e, _N_BINS - e)],
                o_ref.at[pl.ds(r0 + e, _N_BINS - e)], csem))
        if e > s:
            fills.append(pltpu.make_async_copy(
                fillbuf.at[pl.ds(0, e - s)],
                o_ref.at[pl.ds(r0 + s, e - s)], fsem))
    for c in copies:
        c.start()
    for f in fills:
        f.start()
    for c in copies:
        c.wait()
    for f in fills:
        f.wait()


def _kernel_dma(x):
    x2 = x.reshape(_BATCH * _N_BINS, _LENGTH // 128, 128)
    out = pl.pallas_call(
        _dma_body,
        in_specs=[pl.BlockSpec(memory_space=pl.ANY)],
        out_specs=pl.BlockSpec(memory_space=pl.ANY),
        out_shape=jax.ShapeDtypeStruct(
            (_BATCH * _N_BINS, _LENGTH // 128, 128), x.dtype),
        scratch_shapes=[
            pltpu.VMEM((_MAXW, _LENGTH // 128, 128), jnp.float32),
            pltpu.SemaphoreType.DMA,
            pltpu.SemaphoreType.DMA,
        ],
    )(x2)
    return out.reshape(_BATCH, _N_BINS, _LENGTH)


_CH = 32
_NBUF_TC = 6


def _chunks() -> list:
    out = []
    for b in range(_BATCH):
        s, e = int(_START[b]), int(_END[b])
        r0 = b * _N_BINS
        for lo, hi in ((r0, r0 + s), (r0 + e, r0 + _N_BINS)):
            r = lo
            while r < hi:
                n = min(_CH, hi - r)
                out.append((r, n))
                r += n
    return out


_CHUNKS = _chunks()


def _stage_body(x_ref, o_ref, b0, b1, b2, b3, b4, b5, fillbuf,
                i0, i1, i2, i3, i4, i5, o0, o1, o2, o3, o4, o5, fsem):
    bufs = (b0, b1, b2, b3, b4, b5)
    isems = (i0, i1, i2, i3, i4, i5)
    osems = (o0, o1, o2, o3, o4, o5)
    nc = len(_CHUNKS)

    fillbuf[...] = jnp.full((_MAXW, _LENGTH // 128, 128), _MASK_VALUE,
                            jnp.float32)
    fills = []
    for b in range(_BATCH):
        s, e = int(_START[b]), int(_END[b])
        if e > s:
            f = pltpu.make_async_copy(
                fillbuf.at[pl.ds(0, e - s)],
                o_ref.at[pl.ds(b * _N_BINS + s, e - s)], fsem)
            f.start()
            fills.append(f)

    def start_in(i):
        r, n = _CHUNKS[i]
        h = pltpu.make_async_copy(x_ref.at[pl.ds(r, n)],
                                  bufs[i % _NBUF_TC].at[pl.ds(0, n)],
                                  isems[i % _NBUF_TC])
        h.start()
        return h

    in_h = {}
    out_h = {}
    for i in range(min(_NBUF_TC - 2, nc)):
        in_h[i] = start_in(i)
    for i in range(nc):
        b = i % _NBUF_TC
        in_h.pop(i).wait()
        r, n = _CHUNKS[i]
        h = pltpu.make_async_copy(bufs[b].at[pl.ds(0, n)],
                                  o_ref.at[pl.ds(r, n)], osems[b])
        h.start()
        out_h[i] = h
        j = i + _NBUF_TC - 2
        if j < nc:
            prev = j - _NBUF_TC
            if prev >= 0:
                out_h.pop(prev).wait()
            in_h[j] = start_in(j)
    for h in out_h.values():
        h.wait()
    for f in fills:
        f.wait()


def _kernel_stage(x):
    x2 = x.reshape(_BATCH * _N_BINS, _LENGTH // 128, 128)
    buf = pltpu.VMEM((_CH, _LENGTH // 128, 128), jnp.float32)
    out = pl.pallas_call(
        _stage_body,
        in_specs=[pl.BlockSpec(memory_space=pl.ANY)],
        out_specs=pl.BlockSpec(memory_space=pl.ANY),
        out_shape=jax.ShapeDtypeStruct(
            (_BATCH * _N_BINS, _LENGTH // 128, 128), x.dtype),
        scratch_shapes=(
            [buf] * _NBUF_TC
            + [pltpu.VMEM((_MAXW, _LENGTH // 128, 128), jnp.float32)]
            + [pltpu.SemaphoreType.DMA] * (2 * _NBUF_TC + 1)
        ),
    )(x2)
    return out.reshape(_BATCH, _N_BINS, _LENGTH)


_BB = 2


def _where_kernel(st_ref, en_ref, x_ref, o_ref):
    b = pl.program_id(0)
    row = jax.lax.broadcasted_iota(jnp.int32, (1, _N_BINS, 1), 1)
    for k in range(_BB):
        msk = (row >= st_ref[_BB * b + k]) & (row < en_ref[_BB * b + k])
        o_ref[k:k + 1] = jnp.where(msk, jnp.float32(_MASK_VALUE),
                                   x_ref[k:k + 1])


def _kernel_where(x):
    return pl.pallas_call(
        _where_kernel,
        grid_spec=pltpu.PrefetchScalarGridSpec(
            num_scalar_prefetch=2,
            grid=(_BATCH // _BB,),
            in_specs=[pl.BlockSpec((_BB, _N_BINS, _LENGTH),
                                   lambda b, st, en: (b, 0, 0))],
            out_specs=pl.BlockSpec((_BB, _N_BINS, _LENGTH),
                                   lambda b, st, en: (b, 0, 0)),
        ),
        out_shape=jax.ShapeDtypeStruct((_BATCH, _N_BINS, _LENGTH), x.dtype),
    )(jnp.asarray(_START), jnp.asarray(_END), x)


@jax.jit
def kernel(x):
    return _kernel_where(x)


# speedup vs baseline: 44.7557x; 1.0214x over previous
import functools

import jax
import jax.numpy as jnp
import numpy as np
from jax.experimental import pallas as pl
from jax.experimental.pallas import tpu as pltpu

_BATCH = 64
_N_BINS = 128
_LENGTH = 4096
_MASK_VALUE = -80.0
_MAX_WIDTH = 32


def _rotl(x, r):
    return ((x << np.uint32(r)) | (x >> np.uint32(32 - r))).astype(np.uint32)


def _threefry2x32_pair(k1, k2, c1, c2):
    x = [c1.astype(np.uint32).copy(), c2.astype(np.uint32).copy()]
    rotations = [[13, 15, 26, 6], [17, 29, 16, 24]]
    ks = [np.uint32(k1), np.uint32(k2),
          np.uint32(np.uint32(k1) ^ np.uint32(k2) ^ np.uint32(0x1BD11BDA))]
    x[0] = (x[0] + ks[0]).astype(np.uint32)
    x[1] = (x[1] + ks[1]).astype(np.uint32)
    for i in range(5):
        for r in rotations[i % 2]:
            x[0] = (x[0] + x[1]).astype(np.uint32)
            x[1] = _rotl(x[1], r)
            x[1] = x[1] ^ x[0]
        x[0] = (x[0] + ks[(i + 1) % 3]).astype(np.uint32)
        x[1] = (x[1] + ks[(i + 2) % 3] + np.uint32(i + 1)).astype(np.uint32)
    return x[0], x[1]


def _np_uniform(k1, k2, n, minval, maxval):
    b1, b2 = _threefry2x32_pair(k1, k2, np.zeros(n, np.uint32),
                                np.arange(n, dtype=np.uint32))
    bits = b1 ^ b2
    fb = (bits >> np.uint32(9)) | np.uint32(0x3F800000)
    floats = fb.view(np.float32) - np.float32(1.0)
    r = (floats * np.float32(maxval - minval) + np.float32(minval)).astype(np.float32)
    return np.maximum(np.float32(minval), r)


def _mask_bounds() -> tuple[np.ndarray, np.ndarray]:
    b1, b2 = _threefry2x32_pair(np.uint32(0), np.uint32(42),
                                np.zeros(2, np.uint32),
                                np.arange(2, dtype=np.uint32))
    width = _np_uniform(b1[0], b2[0], _BATCH, 0.0, float(_MAX_WIDTH))
    ix = _np_uniform(b1[1], b2[1], _BATCH, 0.0, float(_N_BINS - _MAX_WIDTH))
    start = np.floor(ix).astype(np.int32)
    end = np.floor((ix + width).astype(np.float32)).astype(np.int32)
    return start, end


_START, _END = _mask_bounds()


def _row_lists():
    bins = np.arange(_N_BINS, dtype=np.int32)
    mask = ((bins[None, :] >= _START[:, None]) &
            (bins[None, :] < _END[:, None])).reshape(-1)
    rows = np.arange(_BATCH * _N_BINS, dtype=np.int32)

    def pad_split(r, k):
        per = -(-len(r) // (_NW * k)) * k
        total = per * _NW
        padded = np.concatenate([r, r[:total - len(r)]])
        return padded.reshape(_NW, per // k, k).astype(np.int32)

    return pad_split(rows[~mask], _K), pad_split(rows[mask], _KF)


_NW = 32
_NSUB = 16
_K = 6
_KF = 4
_NBUF = 3
_CIDX, _FIDX = _row_lists()
_GC = _CIDX.shape[1]
_GM = _FIDX.shape[1]


def _sc_body(x_ref, cidx_hbm, fidx_hbm, out_ref,
             cidx_v, fidx_v, buf0, buf1, buf2, fillbuf,
             gsem0, gsem1, gsem2, ssem0, ssem1, ssem2, fsem):
    from jax import lax
    wid = lax.axis_index("c") * _NSUB + lax.axis_index("s")
    pltpu.sync_copy(cidx_hbm.at[wid], cidx_v)
    pltpu.sync_copy(fidx_hbm.at[wid], fidx_v)

    n16 = _LENGTH // 16

    def fill_row(t, c):
        i = t // n16
        j = t % n16
        fillbuf[i, pl.ds(pl.multiple_of(j * 16, 16), 16)] = jnp.full(
            (16,), _MASK_VALUE, jnp.float32)
        return c

    lax.fori_loop(0, _KF * n16, fill_row, 0)

    fills = [pltpu.async_copy(fillbuf, out_ref.at[fidx_v.at[g]], fsem)
             for g in range(_GM)]

    bufs = (buf0, buf1, buf2)
    gsems = (gsem0, gsem1, gsem2)
    ssems = (ssem0, ssem1, ssem2)
    gathers = [None, None, None]
    scatters = [None, None, None]
    gathers[0] = pltpu.async_copy(x_ref.at[cidx_v.at[0]], bufs[0], gsems[0])
    for g in range(_GC):
        b = g % _NBUF
        if g + 1 < _GC:
            nb = (g + 1) % _NBUF
            if scatters[nb] is not None:
                scatters[nb].wait()
                scatters[nb] = None
            gathers[nb] = pltpu.async_copy(x_ref.at[cidx_v.at[g + 1]],
                                           bufs[nb], gsems[nb])
        gathers[b].wait()
        scatters[b] = pltpu.async_copy(bufs[b], out_ref.at[cidx_v.at[g]],
                                       ssems[b])
    for s in scatters:
        if s is not None:
            s.wait()
    for f in fills:
        f.wait()


def _kernel_sc(x):
    from jax.experimental.pallas import tpu_sc as plsc
    mesh = plsc.VectorSubcoreMesh(core_axis_name="c", subcore_axis_name="s")
    x2 = x.reshape(_BATCH * _N_BINS, _LENGTH)
    run = functools.partial(
        pl.kernel,
        out_type=jax.ShapeDtypeStruct((_BATCH * _N_BINS, _LENGTH), x.dtype),
        mesh=mesh,
        scratch_types=[
            pltpu.VMEM((_GC, _K), jnp.int32),
            pltpu.VMEM((_GM, _KF), jnp.int32),
            pltpu.VMEM((_K, _LENGTH), jnp.float32),
            pltpu.VMEM((_K, _LENGTH), jnp.float32),
            pltpu.VMEM((_K, _LENGTH), jnp.float32),
            pltpu.VMEM((_KF, _LENGTH), jnp.float32),
            pltpu.SemaphoreType.DMA,
            pltpu.SemaphoreType.DMA,
            pltpu.SemaphoreType.DMA,
            pltpu.SemaphoreType.DMA,
            pltpu.SemaphoreType.DMA,
            pltpu.SemaphoreType.DMA,
            pltpu.SemaphoreType.DMA,
        ],
    )(_sc_body)
    out = run(x2, jnp.asarray(_CIDX), jnp.asarray(_FIDX))
    return out.reshape(_BATCH, _N_BINS, _LENGTH)


_GROUP = 8
_NGRP = _N_BINS // _GROUP


def _src_groups() -> np.ndarray:
    src = np.empty(_BATCH * _NGRP, np.int32)
    for b in range(_BATCH):
        s, e = int(_START[b]), int(_END[b])
        prev = None
        for g in range(_NGRP):
            lo, hi = g * _GROUP, (g + 1) * _GROUP
            full = s <= lo and hi <= e
            if not full:
                mine = b * _NGRP + g
            elif prev is not None:
                mine = prev
            else:
                mine = b * _NGRP + _NGRP - 1
            src[b * _NGRP + g] = mine
            prev = mine
    return src


def _mask_kernel(srctab_ref, st_ref, en_ref, x_ref, o_ref):
    i = pl.program_id(0)
    b = i // _NGRP
    g = i % _NGRP
    row = g * _GROUP + jax.lax.broadcasted_iota(jnp.int32, (1, _GROUP, 1), 1)
    msk = (row >= st_ref[b]) & (row < en_ref[b])
    o_ref[...] = jnp.where(msk, jnp.float32(_MASK_VALUE), x_ref[...])


_SRCTAB = _src_groups()


def _kernel_tc(x):
    x2 = x.reshape(_BATCH * _NGRP, _GROUP, _LENGTH)
    out = pl.pallas_call(
        _mask_kernel,
        grid_spec=pltpu.PrefetchScalarGridSpec(
            num_scalar_prefetch=3,
            grid=(_BATCH * _NGRP,),
            in_specs=[pl.BlockSpec((1, _GROUP, _LENGTH),
                                   lambda i, tab, st, en: (tab[i], 0, 0))],
            out_specs=pl.BlockSpec((1, _GROUP, _LENGTH),
                                   lambda i, tab, st, en: (i, 0, 0)),
        ),
        out_shape=jax.ShapeDtypeStruct((_BATCH * _NGRP, _GROUP, _LENGTH),
                                       x.dtype),
    )(jnp.asarray(_SRCTAB), jnp.asarray(_START), jnp.asarray(_END), x2)
    return out.reshape(_BATCH, _N_BINS, _LENGTH)


_MAXW = 32


def _dma_body(x_ref, o_ref, fillbuf, csem, fsem):
    fillbuf[...] = jnp.full((_MAXW, _LENGTH // 128, 128), _MASK_VALUE,
                            jnp.float32)
    copies = []
    fills = []
    for b in range(_BATCH):
        s, e = int(_START[b]), int(_END[b])
        r0 = b * _N_BINS
        if s > 0:
            copies.append(pltpu.make_async_copy(
                x_ref.at[pl.ds(r0, s)], o_ref.at[pl.ds(r0, s)], csem))
        if e < _N_BINS:
            copies.append(pltpu.make_async_copy(
                x_ref.at[pl.ds(r0 + e, _N_BINS - e)],
                o_ref.at[pl.ds(r0 + e, _N_BINS - e)], csem))
        if e > s:
            fills.append(pltpu.make_async_copy(
                fillbuf.at[pl.ds(0, e - s)],
                o_ref.at[pl.ds(r0 + s, e - s)], fsem))
    for c in copies:
        c.start()
    for f in fills:
        f.start()
    for c in copies:
        c.wait()
    for f in fills:
        f.wait()


def _kernel_dma(x):
    x2 = x.reshape(_BATCH * _N_BINS, _LENGTH // 128, 128)
    out = pl.pallas_call(
        _dma_body,
        in_specs=[pl.BlockSpec(memory_space=pl.ANY)],
        out_specs=pl.BlockSpec(memory_space=pl.ANY),
        out_shape=jax.ShapeDtypeStruct(
            (_BATCH * _N_BINS, _LENGTH // 128, 128), x.dtype),
        scratch_shapes=[
            pltpu.VMEM((_MAXW, _LENGTH // 128, 128), jnp.float32),
            pltpu.SemaphoreType.DMA,
            pltpu.SemaphoreType.DMA,
        ],
    )(x2)
    return out.reshape(_BATCH, _N_BINS, _LENGTH)


_CH = 32
_NBUF_TC = 6


def _chunks() -> list:
    out = []
    for b in range(_BATCH):
        s, e = int(_START[b]), int(_END[b])
        r0 = b * _N_BINS
        for lo, hi in ((r0, r0 + s), (r0 + e, r0 + _N_BINS)):
            r = lo
            while r < hi:
                n = min(_CH, hi - r)
                out.append((r, n))
                r += n
    return out


_CHUNKS = _chunks()


def _stage_body(x_ref, o_ref, b0, b1, b2, b3, b4, b5, fillbuf,
                i0, i1, i2, i3, i4, i5, o0, o1, o2, o3, o4, o5, fsem):
    bufs = (b0, b1, b2, b3, b4, b5)
    isems = (i0, i1, i2, i3, i4, i5)
    osems = (o0, o1, o2, o3, o4, o5)
    nc = len(_CHUNKS)

    fillbuf[...] = jnp.full((_MAXW, _LENGTH // 128, 128), _MASK_VALUE,
                            jnp.float32)
    fills = []
    for b in range(_BATCH):
        s, e = int(_START[b]), int(_END[b])
        if e > s:
            f = pltpu.make_async_copy(
                fillbuf.at[pl.ds(0, e - s)],
                o_ref.at[pl.ds(b * _N_BINS + s, e - s)], fsem)
            f.start()
            fills.append(f)

    def start_in(i):
        r, n = _CHUNKS[i]
        h = pltpu.make_async_copy(x_ref.at[pl.ds(r, n)],
                                  bufs[i % _NBUF_TC].at[pl.ds(0, n)],
                                  isems[i % _NBUF_TC])
        h.start()
        return h

    in_h = {}
    out_h = {}
    for i in range(min(_NBUF_TC - 2, nc)):
        in_h[i] = start_in(i)
    for i in range(nc):
        b = i % _NBUF_TC
        in_h.pop(i).wait()
        r, n = _CHUNKS[i]
        h = pltpu.make_async_copy(bufs[b].at[pl.ds(0, n)],
                                  o_ref.at[pl.ds(r, n)], osems[b])
        h.start()
        out_h[i] = h
        j = i + _NBUF_TC - 2
        if j < nc:
            prev = j - _NBUF_TC
            if prev >= 0:
                out_h.pop(prev).wait()
            in_h[j] = start_in(j)
    for h in out_h.values():
        h.wait()
    for f in fills:
        f.wait()


def _kernel_stage(x):
    x2 = x.reshape(_BATCH * _N_BINS, _LENGTH // 128, 128)
    buf = pltpu.VMEM((_CH, _LENGTH // 128, 128), jnp.float32)
    out = pl.pallas_call(
        _stage_body,
        in_specs=[pl.BlockSpec(memory_space=pl.ANY)],
        out_specs=pl.BlockSpec(memory_space=pl.ANY),
        out_shape=jax.ShapeDtypeStruct(
            (_BATCH * _N_BINS, _LENGTH // 128, 128), x.dtype),
        scratch_shapes=(
            [buf] * _NBUF_TC
            + [pltpu.VMEM((_MAXW, _LENGTH // 128, 128), jnp.float32)]
            + [pltpu.SemaphoreType.DMA] * (2 * _NBUF_TC + 1)
        ),
    )(x2)
    return out.reshape(_BATCH, _N_BINS, _LENGTH)


_BB = 4


def _where_kernel(st_ref, en_ref, x_ref, o_ref):
    b = pl.program_id(0)
    row = jax.lax.broadcasted_iota(jnp.int32, (1, _N_BINS, 1), 1)
    for k in range(_BB):
        msk = (row >= st_ref[_BB * b + k]) & (row < en_ref[_BB * b + k])
        o_ref[k:k + 1] = jnp.where(msk, jnp.float32(_MASK_VALUE),
                                   x_ref[k:k + 1])


def _kernel_where(x):
    return pl.pallas_call(
        _where_kernel,
        grid_spec=pltpu.PrefetchScalarGridSpec(
            num_scalar_prefetch=2,
            grid=(_BATCH // _BB,),
            in_specs=[pl.BlockSpec((_BB, _N_BINS, _LENGTH),
                                   lambda b, st, en: (b, 0, 0))],
            out_specs=pl.BlockSpec((_BB, _N_BINS, _LENGTH),
                                   lambda b, st, en: (b, 0, 0)),
        ),
        out_shape=jax.ShapeDtypeStruct((_BATCH, _N_BINS, _LENGTH), x.dtype),
    )(jnp.asarray(_START), jnp.asarray(_END), x)


@jax.jit
def kernel(x):
    return _kernel_where(x)
